# Initial kernel scaffold; baseline (speedup 1.0000x reference)
import jax
import jax.numpy as jnp
from jax.experimental import pallas as pl


def kernel(x, W, b, Wri, bri, Wrt, brt, A1, B1, A2, B2, A3, B3, A4, B4):
    raise NotImplementedError("write your pallas kernel here")


# speedup vs baseline: 1.9695x; 1.9695x over previous
import jax
import jax.numpy as jnp
from jax.experimental import pallas as pl
from jax.experimental.pallas import tpu as pltpu

_B, _S, _DIN, _DOUT, _R, _E, _SPLIT = 4, 2048, 2048, 2048, 16, 4, 32
_SCALING = 32.0 / 16.0
_M = _B * _S
_TM = 512


def _body(x_ref, wt_ref, side_ref, bct_ref, b_ref, sb_ref, o_ref):
    m = pl.program_id(0)
    xb = x_ref[:].astype(jnp.bfloat16)
    acc = jnp.dot(xb, wt_ref[:], preferred_element_type=jnp.float32)
    side = jnp.dot(xb, side_ref[:], preferred_element_type=jnp.float32)
    side = side + sb_ref[:]
    h = side[:, :64]
    logits = side[:, 64:72]
    row = jax.lax.broadcasted_iota(jnp.int32, (_TM, 1), 0) + m * _TM
    is_img = (row % _S) < _SPLIT
    sel = jnp.where(is_img, logits[:, :4], logits[:, 4:8])
    sel = sel - jnp.max(sel, axis=1, keepdims=True)
    e = jnp.exp(sel)
    gate = e / jnp.sum(e, axis=1, keepdims=True)
    ghat = jnp.concatenate(
        [gate[:, i : i + 1] * h[:, i * _R : (i + 1) * _R] for i in range(_E)],
        axis=1,
    ) * _SCALING
    lora = jnp.dot(ghat.astype(jnp.bfloat16), bct_ref[:],
                   preferred_element_type=jnp.float32)
    o_ref[:] = acc + lora + b_ref[:]


def kernel(x, W, b, Wri, bri, Wrt, brt, A1, B1, A2, B2, A3, B3, A4, B4):
    xf = x.reshape(_M, _DIN)
    wt = W.T.astype(jnp.bfloat16)
    side = jnp.concatenate([A1, A2, A3, A4, Wri, Wrt], axis=0)
    side = jnp.pad(side, ((0, 128 - 72), (0, 0))).T.astype(jnp.bfloat16)
    bct = jnp.concatenate([B1, B2, B3, B4], axis=1).T.astype(jnp.bfloat16)
    bias = b.reshape(1, _DOUT).astype(jnp.float32)
    sbias = jnp.zeros((1, 128), jnp.float32)
    sbias = sbias.at[0, 64:68].set(bri).at[0, 68:72].set(brt)

    out = pl.pallas_call(
        _body,
        grid=(_M // _TM,),
        in_specs=[
            pl.BlockSpec((_TM, _DIN), lambda m: (m, 0)),
            pl.BlockSpec((_DIN, _DOUT), lambda m: (0, 0)),
            pl.BlockSpec((_DIN, 128), lambda m: (0, 0)),
            pl.BlockSpec((64, _DOUT), lambda m: (0, 0)),
            pl.BlockSpec((1, _DOUT), lambda m: (0, 0)),
            pl.BlockSpec((1, 128), lambda m: (0, 0)),
        ],
        out_specs=pl.BlockSpec((_TM, _DOUT), lambda m: (m, 0)),
        out_shape=jax.ShapeDtypeStruct((_M, _DOUT), jnp.float32),
        compiler_params=pltpu.CompilerParams(
            dimension_semantics=("arbitrary",),
        ),
    )(xf, wt, side, bct, bias, sbias)
    return out.reshape(_B, _S, _DOUT)


# speedup vs baseline: 2.3808x; 1.2088x over previous
import jax
import jax.numpy as jnp
from jax.experimental import pallas as pl
from jax.experimental.pallas import tpu as pltpu

_B, _S, _DIN, _DOUT, _R, _E, _SPLIT = 4, 2048, 2048, 2048, 16, 4, 32
_SCALING = 32.0 / 16.0
_M = _B * _S
_TM = 1024
_KX = _DIN + 128


def _body(x_ref, rhs_ref, side_ref, b_ref, sb_ref, o_ref, lhs_ref):
    m = pl.program_id(0)
    xb = x_ref[:].astype(jnp.bfloat16)
    lhs_ref[:, :_DIN] = xb
    side = jnp.dot(xb, side_ref[:], preferred_element_type=jnp.float32)
    side = side + sb_ref[:]
    h = side[:, :64]
    logits = side[:, 64:72]
    row = jax.lax.broadcasted_iota(jnp.int32, (_TM, 1), 0) + m * _TM
    is_img = (row % _S) < _SPLIT
    sel = jnp.where(is_img, logits[:, :4], logits[:, 4:8])
    sel = sel - jnp.max(sel, axis=1, keepdims=True)
    e = jnp.exp(sel)
    gate = e / jnp.sum(e, axis=1, keepdims=True)
    gh = jnp.concatenate(
        [gate[:, i : i + 1] * h[:, i * _R : (i + 1) * _R] for i in range(_E)]
        + [jnp.zeros((_TM, 64), jnp.float32)],
        axis=1,
    )
    lhs_ref[:, _DIN:] = gh.astype(jnp.bfloat16)
    acc = jax.lax.dot_general(lhs_ref[:], rhs_ref[:], (((1,), (1,)), ((), ())),
                              preferred_element_type=jnp.float32)
    o_ref[:] = acc + b_ref[:]


def kernel(x, W, b, Wri, bri, Wrt, brt, A1, B1, A2, B2, A3, B3, A4, B4):
    xf = x.reshape(_M, _DIN)
    rhs = jnp.concatenate(
        [W, jnp.concatenate([B1, B2, B3, B4], axis=1) * _SCALING,
         jnp.zeros((_DOUT, 64), jnp.float32)], axis=1).astype(jnp.bfloat16)
    side = jnp.concatenate([A1, A2, A3, A4, Wri, Wrt], axis=0)
    side = jnp.pad(side, ((0, 128 - 72), (0, 0))).T.astype(jnp.bfloat16)
    bias = b.reshape(1, _DOUT).astype(jnp.float32)
    sbias = jnp.pad(jnp.concatenate([bri, brt]).reshape(1, 8),
                    ((0, 0), (64, 56))).astype(jnp.float32)

    out = pl.pallas_call(
        _body,
        grid=(_M // _TM,),
        in_specs=[
            pl.BlockSpec((_TM, _DIN), lambda m: (m, 0)),
            pl.BlockSpec((_DOUT, _KX), lambda m: (0, 0)),
            pl.BlockSpec((_DIN, 128), lambda m: (0, 0)),
            pl.BlockSpec((1, _DOUT), lambda m: (0, 0)),
            pl.BlockSpec((1, 128), lambda m: (0, 0)),
        ],
        out_specs=pl.BlockSpec((_TM, _DOUT), lambda m: (m, 0)),
        out_shape=jax.ShapeDtypeStruct((_M, _DOUT), jnp.float32),
        scratch_shapes=[pltpu.VMEM((_TM, _KX), jnp.bfloat16)],
        compiler_params=pltpu.CompilerParams(
            dimension_semantics=("arbitrary",),
        ),
    )(xf, rhs, side, bias, sbias)
    return out.reshape(_B, _S, _DOUT)


# speedup vs baseline: 2.3842x; 1.0014x over previous
import jax
import jax.numpy as jnp
from jax.experimental import pallas as pl
from jax.experimental.pallas import tpu as pltpu

_B, _S, _DIN, _DOUT, _R, _E, _SPLIT = 4, 2048, 2048, 2048, 16, 4, 32
_SCALING = 32.0 / 16.0
_M = _B * _S
_TM = 1024
_KX = _DIN + 128


def _body(x_ref, rhs_ref, side_ref, b_ref, sb_ref, o_ref, lhs_ref):
    m = pl.program_id(0)
    xb = x_ref[:].astype(jnp.bfloat16)
    lhs_ref[:, :_DIN] = xb
    side = jnp.dot(xb, side_ref[:], preferred_element_type=jnp.float32)
    side = side + sb_ref[:]
    h = side[:, :64]
    logits = side[:, 64:72]
    row = jax.lax.broadcasted_iota(jnp.int32, (_TM, 1), 0) + m * _TM
    is_img = (row % _S) < _SPLIT
    sel = jnp.where(is_img, logits[:, :4], logits[:, 4:8])
    sel = sel - jnp.max(sel, axis=1, keepdims=True)
    e = jnp.exp(sel)
    gate = e / jnp.sum(e, axis=1, keepdims=True)
    gh = jnp.concatenate(
        [gate[:, i : i + 1] * h[:, i * _R : (i + 1) * _R] for i in range(_E)]
        + [jnp.zeros((_TM, 64), jnp.float32)],
        axis=1,
    )
    lhs_ref[:, _DIN:] = gh.astype(jnp.bfloat16)
    acc = jax.lax.dot_general(lhs_ref[:], rhs_ref[:], (((1,), (1,)), ((), ())),
                              preferred_element_type=jnp.float32)
    o_ref[:] = acc + b_ref[:]


def kernel(x, W, b, Wri, bri, Wrt, brt, A1, B1, A2, B2, A3, B3, A4, B4):
    xf = x.reshape(_M, _DIN)
    rhs = jnp.concatenate(
        [W.astype(jnp.bfloat16),
         (jnp.concatenate([B1, B2, B3, B4], axis=1) * _SCALING).astype(jnp.bfloat16),
         jnp.zeros((_DOUT, 64), jnp.bfloat16)], axis=1)
    side = jnp.concatenate([A1, A2, A3, A4, Wri, Wrt], axis=0)
    side = jnp.pad(side, ((0, 128 - 72), (0, 0))).T.astype(jnp.bfloat16)
    bias = b.reshape(1, _DOUT).astype(jnp.float32)
    sbias = jnp.pad(jnp.concatenate([bri, brt]).reshape(1, 8),
                    ((0, 0), (64, 56))).astype(jnp.float32)

    out = pl.pallas_call(
        _body,
        grid=(_M // _TM,),
        in_specs=[
            pl.BlockSpec((_TM, _DIN), lambda m: (m, 0)),
            pl.BlockSpec((_DOUT, _KX), lambda m: (0, 0)),
            pl.BlockSpec((_DIN, 128), lambda m: (0, 0)),
            pl.BlockSpec((1, _DOUT), lambda m: (0, 0)),
            pl.BlockSpec((1, 128), lambda m: (0, 0)),
        ],
        out_specs=pl.BlockSpec((_TM, _DOUT), lambda m: (m, 0)),
        out_shape=jax.ShapeDtypeStruct((_M, _DOUT), jnp.float32),
        scratch_shapes=[pltpu.VMEM((_TM, _KX), jnp.bfloat16)],
        compiler_params=pltpu.CompilerParams(
            dimension_semantics=("arbitrary",),
        ),
    )(xf, rhs, side, bias, sbias)
    return out.reshape(_B, _S, _DOUT)
